# Initial kernel scaffold; baseline (speedup 1.0000x reference)
import jax
import jax.numpy as jnp
from jax.experimental import pallas as pl


def kernel(x, edge_index, W1, b1, W2, b2, W3, b3):
    raise NotImplementedError("write your pallas kernel here")


# speedup vs baseline: 6.5127x; 6.5127x over previous
import functools

import jax
import jax.numpy as jnp
from jax import lax
from jax.experimental import pallas as pl
from jax.experimental.pallas import tpu as pltpu
from jax.experimental.pallas import tpu_sc as plsc

N = 10000
E = 320000
D = 128
NC = 2
NS = 16
NW = NC * NS
EPW = E // NW
CHUNK = 125
NCHUNK = EPW // CHUNK
AP = 624
TAIL = N - NS * AP
DW = 16

_mesh = plsc.VectorSubcoreMesh(core_axis_name="c", subcore_axis_name="s")


def _count_body(idxr, zrows, ones_hbm, out_hbm, idx_v, ones_v, acc):
    cid = lax.axis_index("c")
    sid = lax.axis_index("s")
    wid = cid * NS + sid

    pltpu.sync_copy(ones_hbm, ones_v)
    base = sid * AP
    pltpu.sync_copy(zrows, acc.at[pl.ds(base, AP)])

    @pl.when(sid == 0)
    def _():
        pltpu.sync_copy(zrows.at[pl.ds(0, TAIL)], acc.at[pl.ds(NS * AP, TAIL)])

    pltpu.sync_copy(idxr.at[wid], idx_v)
    plsc.subcore_barrier()

    @pl.loop(0, NCHUNK)
    def _(j):
        pltpu.sync_copy(ones_v, acc.at[idx_v.at[j]], add=True)

    plsc.subcore_barrier()
    pltpu.sync_copy(acc.at[pl.ds(base, AP)],
                    out_hbm.at[pl.ds(cid * N + base, AP)])

    @pl.when(sid == 0)
    def _():
        pltpu.sync_copy(acc.at[pl.ds(NS * AP, TAIL)],
                        out_hbm.at[pl.ds(cid * N + NS * AP, TAIL)])


_count = pl.kernel(
    _count_body,
    out_type=jax.ShapeDtypeStruct((NC * N, D), jnp.float32),
    mesh=_mesh,
    scratch_types=[
        pltpu.VMEM((NCHUNK, CHUNK), jnp.int32),
        pltpu.VMEM((CHUNK, D), jnp.float32),
        pltpu.VMEM_SHARED((N, D), jnp.float32),
    ],
)


def _spmm_body(h_hbm, srcr, dstr, zrows, out_hbm,
               src_v, dst_v, rows_v, acc, gsem):
    cid = lax.axis_index("c")
    sid = lax.axis_index("s")
    wid = cid * NS + sid

    base = sid * AP
    pltpu.sync_copy(zrows, acc.at[pl.ds(base, AP)])

    @pl.when(sid == 0)
    def _():
        pltpu.sync_copy(zrows.at[pl.ds(0, TAIL)], acc.at[pl.ds(NS * AP, TAIL)])

    pltpu.sync_copy(srcr.at[wid], src_v)
    pltpu.sync_copy(dstr.at[wid], dst_v)
    plsc.subcore_barrier()

    @pl.loop(0, NCHUNK)
    def _(j):
        pltpu.async_copy(h_hbm.at[src_v.at[j]], rows_v, gsem).wait()
        pltpu.sync_copy(rows_v, acc.at[dst_v.at[j]], add=True)

    plsc.subcore_barrier()
    pltpu.sync_copy(acc.at[pl.ds(base, AP)],
                    out_hbm.at[pl.ds(cid * N + base, AP)])

    @pl.when(sid == 0)
    def _():
        pltpu.sync_copy(acc.at[pl.ds(NS * AP, TAIL)],
                        out_hbm.at[pl.ds(cid * N + NS * AP, TAIL)])


_spmm = pl.kernel(
    _spmm_body,
    out_type=jax.ShapeDtypeStruct((NC * N, D), jnp.float32),
    mesh=_mesh,
    scratch_types=[
        pltpu.VMEM((NCHUNK, CHUNK), jnp.int32),
        pltpu.VMEM((NCHUNK, CHUNK), jnp.int32),
        pltpu.VMEM((CHUNK, D), jnp.float32),
        pltpu.VMEM_SHARED((N, D), jnp.float32),
        pltpu.SemaphoreType.DMA,
    ],
)


_BT = 1000


def _prep_body(x_ref, dop_ref, dip_ref, xs_ref, ns_ref, nd_ref):
    dout = dop_ref[0][:, :DW] + dop_ref[1][:, :DW]
    din = dip_ref[0][:, :DW] + dip_ref[1][:, :DW]
    ns = lax.rsqrt(jnp.maximum(dout, 1.0))
    nd = lax.rsqrt(jnp.maximum(din, 1.0))
    ns_ref[...] = ns
    nd_ref[...] = nd
    xs_ref[...] = x_ref[...] * ns[:, 0:1]


def _layer_body(relu_next, p_ref, ns_ref, nd_ref, w_ref, b_ref, o_ref):
    agg = (p_ref[0] + p_ref[1]) * nd_ref[...][:, 0:1]
    h = jnp.dot(agg, w_ref[...], preferred_element_type=jnp.float32)
    h = h + b_ref[...]
    if relu_next:
        h = jnp.maximum(h, 0.0) * ns_ref[...][:, 0:1]
    o_ref[...] = h


def _prep(x, dout_p, din_p):
    grid = N // _BT
    return pl.pallas_call(
        _prep_body,
        grid=(grid,),
        in_specs=[
            pl.BlockSpec((_BT, D), lambda i: (i, 0)),
            pl.BlockSpec((NC, _BT, D), lambda i: (0, i, 0)),
            pl.BlockSpec((NC, _BT, D), lambda i: (0, i, 0)),
        ],
        out_specs=[
            pl.BlockSpec((_BT, D), lambda i: (i, 0)),
            pl.BlockSpec((_BT, DW), lambda i: (i, 0)),
            pl.BlockSpec((_BT, DW), lambda i: (i, 0)),
        ],
        out_shape=[
            jax.ShapeDtypeStruct((N, D), jnp.float32),
            jax.ShapeDtypeStruct((N, DW), jnp.float32),
            jax.ShapeDtypeStruct((N, DW), jnp.float32),
        ],
    )(x, dout_p, din_p)


def _layer(parts, ns, nd, W, b, relu_next):
    grid = N // _BT
    return pl.pallas_call(
        functools.partial(_layer_body, relu_next),
        grid=(grid,),
        in_specs=[
            pl.BlockSpec((NC, _BT, D), lambda i: (0, i, 0)),
            pl.BlockSpec((_BT, DW), lambda i: (i, 0)),
            pl.BlockSpec((_BT, DW), lambda i: (i, 0)),
            pl.BlockSpec((D, D), lambda i: (0, 0)),
            pl.BlockSpec((1, D), lambda i: (0, 0)),
        ],
        out_specs=pl.BlockSpec((_BT, D), lambda i: (i, 0)),
        out_shape=jax.ShapeDtypeStruct((N, D), jnp.float32),
    )(parts, ns, nd, W, b.reshape(1, D))


def kernel(x, edge_index, W1, b1, W2, b2, W3, b3):
    ei = edge_index.astype(jnp.int32)
    srcr = ei[0].reshape(NW, NCHUNK, CHUNK)
    dstr = ei[1].reshape(NW, NCHUNK, CHUNK)
    zrows = jnp.zeros((AP, D), jnp.float32)
    ones = jnp.ones((CHUNK, D), jnp.float32)

    dout_p = _count(srcr, zrows, ones).reshape(NC, N, D)
    din_p = _count(dstr, zrows, ones).reshape(NC, N, D)
    h, ns, nd = _prep(x, dout_p, din_p)

    for W, b, relu_next in ((W1, b1, True), (W2, b2, True), (W3, b3, False)):
        parts = _spmm(h, srcr, dstr, zrows).reshape(NC, N, D)
        h = _layer(parts, ns, nd, W, b, relu_next)
    return h
